# baseline (device time: 14409 ns/iter reference)
import jax
import jax.numpy as jnp
from jax import lax
from jax.experimental import pallas as pl
from jax.experimental.pallas import tpu as pltpu

N_DEV = 4
MB = 512 // N_DEV
NH = 4


def _gelu(z):
    return 0.5 * z * (1.0 + jnp.tanh(0.7978845608 * (z + 0.044715 * z * z * z)))


def kernel(A, B):
    m, k = A.shape
    _, n = B.shape
    hc = n // NH

    SEND_ORDER = (2, 1, 3)

    def body(a_ref, b_ref, out_ref, a_bf, b_bf, send_buf, rs_ref, ag_buf,
             ag_ref, rs_send_sems, rs_recv_sems, ag_send_sems, ag_recv_sems):
        my_pos = lax.axis_index("i")

        barrier_sem = pltpu.get_barrier_semaphore()
        for d in (1, 3):
            pl.semaphore_signal(
                barrier_sem, inc=1,
                device_id=((my_pos + d) % N_DEV,),
                device_id_type=pl.DeviceIdType.MESH,
            )
        a_bf[:, :] = a_ref[:, :].astype(jnp.bfloat16)
        b_bf[:, :] = b_ref[:, :].astype(jnp.bfloat16)
        pl.semaphore_wait(barrier_sem, 2)

        rs_sends = []
        for h in range(NH):
            for i, d in enumerate(SEND_ORDER):
                t = (my_pos + d) % N_DEV
                blk = jnp.dot(a_bf[pl.ds(t * MB, MB), :],
                              b_bf[:, pl.ds(h * hc, hc)],
                              preferred_element_type=jnp.float32)
                send_buf[i, h, :, :] = blk.astype(jnp.bfloat16)
                rdma = pltpu.make_async_remote_copy(
                    src_ref=send_buf.at[i, h],
                    dst_ref=rs_ref.at[N_DEV - d, h],
                    send_sem=rs_send_sems.at[i, h],
                    recv_sem=rs_recv_sems.at[N_DEV - d, h],
                    device_id=(t,),
                    device_id_type=pl.DeviceIdType.MESH,
                )
                rdma.start()
                rs_sends.append(rdma)

        own = [
            jnp.dot(a_bf[pl.ds(my_pos * MB, MB), :],
                    b_bf[:, pl.ds(h * hc, hc)],
                    preferred_element_type=jnp.float32)
            for h in range(NH)
        ]

        ag_sends = []
        for h in range(NH):
            for s in (1, 2, 3):
                recv = pltpu.make_async_remote_copy(
                    src_ref=rs_ref.at[s, h],
                    dst_ref=rs_ref.at[s, h],
                    send_sem=rs_send_sems.at[0, h],
                    recv_sem=rs_recv_sems.at[s, h],
                    device_id=(my_pos,),
                    device_id_type=pl.DeviceIdType.MESH,
                )
                recv.wait_recv()
            acc = own[h] + (rs_ref[1, h].astype(jnp.float32)
                            + rs_ref[2, h].astype(jnp.float32)
                            + rs_ref[3, h].astype(jnp.float32))
            z = _gelu(acc)
            out_ref[pl.ds(my_pos * MB, MB), pl.ds(h * hc, hc)] = z
            ag_buf[h, :, :] = z.astype(jnp.bfloat16)
            for i, d in enumerate(SEND_ORDER):
                t = (my_pos + d) % N_DEV
                rdma = pltpu.make_async_remote_copy(
                    src_ref=ag_buf.at[h],
                    dst_ref=ag_ref.at[N_DEV - d, h],
                    send_sem=ag_send_sems.at[i, h],
                    recv_sem=ag_recv_sems.at[N_DEV - d, h],
                    device_id=(t,),
                    device_id_type=pl.DeviceIdType.MESH,
                )
                rdma.start()
                ag_sends.append(rdma)

        for h in range(NH):
            for s in (1, 2, 3):
                src_pos = (my_pos + s) % N_DEV
                recv = pltpu.make_async_remote_copy(
                    src_ref=ag_ref.at[s, h],
                    dst_ref=ag_ref.at[s, h],
                    send_sem=ag_send_sems.at[0, h],
                    recv_sem=ag_recv_sems.at[s, h],
                    device_id=(my_pos,),
                    device_id_type=pl.DeviceIdType.MESH,
                )
                recv.wait_recv()
                out_ref[pl.ds(src_pos * MB, MB), pl.ds(h * hc, hc)] = (
                    ag_ref[s, h].astype(jnp.float32)
                )

        for rdma in rs_sends + ag_sends:
            rdma.wait_send()

    return pl.pallas_call(
        body,
        out_shape=jax.ShapeDtypeStruct((m, n), jnp.float32),
        in_specs=[
            pl.BlockSpec(memory_space=pltpu.VMEM),
            pl.BlockSpec(memory_space=pltpu.VMEM),
        ],
        out_specs=pl.BlockSpec(memory_space=pltpu.VMEM),
        scratch_shapes=[
            pltpu.VMEM((m, k), jnp.bfloat16),
            pltpu.VMEM((k, n), jnp.bfloat16),
            pltpu.VMEM((N_DEV - 1, NH, MB, hc), jnp.bfloat16),
            pltpu.VMEM((N_DEV, NH, MB, hc), jnp.bfloat16),
            pltpu.VMEM((NH, MB, hc), jnp.bfloat16),
            pltpu.VMEM((N_DEV, NH, MB, hc), jnp.bfloat16),
            pltpu.SemaphoreType.DMA((N_DEV - 1, NH)),
            pltpu.SemaphoreType.DMA((N_DEV, NH)),
            pltpu.SemaphoreType.DMA((N_DEV - 1, NH)),
            pltpu.SemaphoreType.DMA((N_DEV, NH)),
        ],
        compiler_params=pltpu.CompilerParams(collective_id=0),
    )(A, B)


# device time: 14377 ns/iter; 1.0022x vs baseline; 1.0022x over previous
import jax
import jax.numpy as jnp
from jax import lax
from jax.experimental import pallas as pl
from jax.experimental.pallas import tpu as pltpu

N_DEV = 4
MB = 512 // N_DEV
NH = 2


def _gelu(z):
    return 0.5 * z * (1.0 + jnp.tanh(0.7978845608 * (z + 0.044715 * z * z * z)))


def kernel(A, B):
    m, k = A.shape
    _, n = B.shape
    hc = n // NH

    SEND_ORDER = (2, 1, 3)

    def body(a_ref, b_ref, out_ref, a_bf, b_bf, send_buf, rs_ref, ag_buf,
             ag_ref, rs_send_sems, rs_recv_sems, ag_send_sems, ag_recv_sems):
        my_pos = lax.axis_index("i")

        barrier_sem = pltpu.get_barrier_semaphore()
        for d in (1, 3):
            pl.semaphore_signal(
                barrier_sem, inc=1,
                device_id=((my_pos + d) % N_DEV,),
                device_id_type=pl.DeviceIdType.MESH,
            )
        a_bf[:, :] = a_ref[:, :].astype(jnp.bfloat16)
        b_bf[:, :] = b_ref[:, :].astype(jnp.bfloat16)
        pl.semaphore_wait(barrier_sem, 2)

        rs_sends = []
        for h in range(NH):
            for i, d in enumerate(SEND_ORDER):
                t = (my_pos + d) % N_DEV
                blk = jnp.dot(a_bf[pl.ds(t * MB, MB), :],
                              b_bf[:, pl.ds(h * hc, hc)],
                              preferred_element_type=jnp.float32)
                send_buf[i, h, :, :] = blk.astype(jnp.bfloat16)
                rdma = pltpu.make_async_remote_copy(
                    src_ref=send_buf.at[i, h],
                    dst_ref=rs_ref.at[N_DEV - d, h],
                    send_sem=rs_send_sems.at[i, h],
                    recv_sem=rs_recv_sems.at[N_DEV - d, h],
                    device_id=(t,),
                    device_id_type=pl.DeviceIdType.MESH,
                )
                rdma.start()
                rs_sends.append(rdma)

        own = [
            jnp.dot(a_bf[pl.ds(my_pos * MB, MB), :],
                    b_bf[:, pl.ds(h * hc, hc)],
                    preferred_element_type=jnp.float32)
            for h in range(NH)
        ]

        ag_sends = []
        for h in range(NH):
            for s in (1, 2, 3):
                recv = pltpu.make_async_remote_copy(
                    src_ref=rs_ref.at[s, h],
                    dst_ref=rs_ref.at[s, h],
                    send_sem=rs_send_sems.at[0, h],
                    recv_sem=rs_recv_sems.at[s, h],
                    device_id=(my_pos,),
                    device_id_type=pl.DeviceIdType.MESH,
                )
                recv.wait_recv()
            acc = own[h] + (rs_ref[1, h].astype(jnp.float32)
                            + rs_ref[2, h].astype(jnp.float32)
                            + rs_ref[3, h].astype(jnp.float32))
            z = _gelu(acc)
            out_ref[pl.ds(my_pos * MB, MB), pl.ds(h * hc, hc)] = z
            ag_buf[h, :, :] = z.astype(jnp.bfloat16)
            for i, d in enumerate(SEND_ORDER):
                t = (my_pos + d) % N_DEV
                rdma = pltpu.make_async_remote_copy(
                    src_ref=ag_buf.at[h],
                    dst_ref=ag_ref.at[N_DEV - d, h],
                    send_sem=ag_send_sems.at[i, h],
                    recv_sem=ag_recv_sems.at[N_DEV - d, h],
                    device_id=(t,),
                    device_id_type=pl.DeviceIdType.MESH,
                )
                rdma.start()
                ag_sends.append(rdma)

        for h in range(NH):
            for s in (1, 2, 3):
                src_pos = (my_pos + s) % N_DEV
                recv = pltpu.make_async_remote_copy(
                    src_ref=ag_ref.at[s, h],
                    dst_ref=ag_ref.at[s, h],
                    send_sem=ag_send_sems.at[0, h],
                    recv_sem=ag_recv_sems.at[s, h],
                    device_id=(my_pos,),
                    device_id_type=pl.DeviceIdType.MESH,
                )
                recv.wait_recv()
                out_ref[pl.ds(src_pos * MB, MB), pl.ds(h * hc, hc)] = (
                    ag_ref[s, h].astype(jnp.float32)
                )

        for rdma in rs_sends + ag_sends:
            rdma.wait_send()

    return pl.pallas_call(
        body,
        out_shape=jax.ShapeDtypeStruct((m, n), jnp.float32),
        in_specs=[
            pl.BlockSpec(memory_space=pltpu.VMEM),
            pl.BlockSpec(memory_space=pltpu.VMEM),
        ],
        out_specs=pl.BlockSpec(memory_space=pltpu.VMEM),
        scratch_shapes=[
            pltpu.VMEM((m, k), jnp.bfloat16),
            pltpu.VMEM((k, n), jnp.bfloat16),
            pltpu.VMEM((N_DEV - 1, NH, MB, hc), jnp.bfloat16),
            pltpu.VMEM((N_DEV, NH, MB, hc), jnp.bfloat16),
            pltpu.VMEM((NH, MB, hc), jnp.bfloat16),
            pltpu.VMEM((N_DEV, NH, MB, hc), jnp.bfloat16),
            pltpu.SemaphoreType.DMA((N_DEV - 1, NH)),
            pltpu.SemaphoreType.DMA((N_DEV, NH)),
            pltpu.SemaphoreType.DMA((N_DEV - 1, NH)),
            pltpu.SemaphoreType.DMA((N_DEV, NH)),
        ],
        compiler_params=pltpu.CompilerParams(collective_id=0),
    )(A, B)
